# baseline (speedup 1.0000x reference)
import functools

import jax
import jax.numpy as jnp
from jax import lax
from jax.experimental import pallas as pl
from jax.experimental.pallas import tpu as pltpu
from jax.experimental.pallas import tpu_sc as plsc

L = 16
NC = 2
NS = 16
NW = NC * NS


def _build(rows, feats):
    rpw = rows // NW
    rb = 8
    nb = rpw // rb
    nch = feats // L

    mesh = plsc.VectorSubcoreMesh(core_axis_name="c", subcore_axis_name="s")

    @functools.partial(
        pl.kernel,
        out_type=(
            jax.ShapeDtypeStruct((rows, feats), jnp.float32),
            jax.ShapeDtypeStruct((rows,), jnp.float32),
        ),
        mesh=mesh,
        scratch_types=[
            pltpu.VMEM((rb, feats), jnp.float32),
            pltpu.VMEM((rb, feats), jnp.float32),
            pltpu.VMEM((rpw,), jnp.float32),
        ],
    )
    def rev_kernel(x_hbm, y_hbm, ld_hbm, in_v, out_v, zeros_v):
        wid = lax.axis_index("s") * NC + lax.axis_index("c")
        base = wid * rpw

        zv = jnp.zeros((L,), jnp.float32)

        def zbody(i, carry):
            zeros_v[pl.ds(i * L, L)] = zv
            return carry

        lax.fori_loop(0, rpw // L, zbody, 0)
        pltpu.sync_copy(zeros_v, ld_hbm.at[pl.ds(base, rpw)])

        def block(g, carry):
            row0 = base + g * rb
            pltpu.sync_copy(x_hbm.at[pl.ds(row0, rb)], in_v)
            for r in range(rb):
                def chunk(j, c2):
                    v = in_v[r, pl.ds((nch - 1 - j) * L, L)]
                    out_v[r, pl.ds(j * L, L)] = lax.rev(v, (0,))
                    return c2

                lax.fori_loop(0, nch, chunk, 0)
            pltpu.sync_copy(out_v, y_hbm.at[pl.ds(row0, rb)])
            return carry

        lax.fori_loop(0, nb, block, 0)

    return rev_kernel


def kernel(x, perm):
    rows, feats = x.shape
    y, logdet = _build(rows, feats)(x)
    return (y, logdet)


# speedup vs baseline: 1.1474x; 1.1474x over previous
import functools

import jax
import jax.numpy as jnp
from jax import lax
from jax.experimental import pallas as pl
from jax.experimental.pallas import tpu as pltpu
from jax.experimental.pallas import tpu_sc as plsc

L = 16
NC = 2
NS = 16
NW = NC * NS


def _build(rows, feats):
    rpw = rows // NW
    rb = 4
    nb = rpw // rb
    nch = feats // L
    blk = rb * feats
    cpb = rb * nch

    mesh = plsc.VectorSubcoreMesh(core_axis_name="c", subcore_axis_name="s")

    @functools.partial(
        pl.kernel,
        out_type=(
            jax.ShapeDtypeStruct((rows * feats,), jnp.float32),
            jax.ShapeDtypeStruct((rows,), jnp.float32),
        ),
        mesh=mesh,
        scratch_types=[
            pltpu.VMEM((2, blk), jnp.float32),
            pltpu.VMEM((2, blk), jnp.float32),
            pltpu.VMEM((rpw,), jnp.float32),
            pltpu.SemaphoreType.DMA,
            pltpu.SemaphoreType.DMA,
            pltpu.SemaphoreType.DMA,
            pltpu.SemaphoreType.DMA,
        ],
    )
    def rev_kernel(x_hbm, y_hbm, ld_hbm, in_v, out_v, zeros_v,
                   sin0, sin1, sout0, sout1):
        wid = lax.axis_index("s") * NC + lax.axis_index("c")
        base = wid * rpw * feats
        sins = (sin0, sin1)
        souts = (sout0, sout1)

        zv = jnp.zeros((L,), jnp.float32)

        @plsc.parallel_loop(0, rpw // L)
        def _zfill(i):
            zeros_v[pl.ds(i * L, L)] = zv

        pltpu.sync_copy(zeros_v, ld_hbm.at[pl.ds(wid * rpw, rpw)])

        def in_copy(g, b):
            return pltpu.async_copy(
                x_hbm.at[pl.ds(base + g * blk, blk)], in_v.at[b], sins[b])

        def out_copy(g, b):
            return pltpu.async_copy(
                out_v.at[b], y_hbm.at[pl.ds(base + g * blk, blk)], souts[b])

        in_copy(0, 0)

        @pl.loop(0, nb, step=2)
        def _blocks(g0):
            for b in range(2):
                g = g0 + b
                bn = (b + 1) % 2

                @pl.when(g + 1 < nb)
                def _prefetch():
                    in_copy(g + 1, bn)

                pltpu.make_async_copy(
                    x_hbm.at[pl.ds(base + g * blk, blk)],
                    in_v.at[b], sins[b]).wait()

                @pl.when(g >= 2)
                def _drain():
                    pltpu.make_async_copy(
                        out_v.at[b],
                        y_hbm.at[pl.ds(base + g * blk, blk)],
                        souts[b]).wait()

                @plsc.parallel_loop(0, cpb, unroll=8)
                def _chunk(k):
                    src = k ^ (nch - 1)
                    v = in_v[b, pl.ds(src * L, L)]
                    out_v[b, pl.ds(k * L, L)] = lax.rev(v, (0,))

                out_copy(g, b)

        for b in range(2):
            pltpu.make_async_copy(
                out_v.at[b],
                y_hbm.at[pl.ds(base + (nb - 2 + b) * blk, blk)],
                souts[b]).wait()

    return rev_kernel


def kernel(x, perm):
    rows, feats = x.shape
    yf, logdet = _build(rows, feats)(x.reshape(-1))
    return (yf.reshape(rows, feats), logdet)


# speedup vs baseline: 3.7302x; 3.2511x over previous
import functools

import jax
import jax.numpy as jnp
from jax import lax
from jax.experimental import pallas as pl
from jax.experimental.pallas import tpu as pltpu
from jax.experimental.pallas import tpu_sc as plsc

L = 16
NC = 2
NS = 16
NW = NC * NS


def _build(rows, feats):
    rpw = rows // NW
    rb = 4
    nb = rpw // rb
    nch = feats // L

    mesh = plsc.VectorSubcoreMesh(core_axis_name="c", subcore_axis_name="s")

    @functools.partial(
        pl.kernel,
        out_type=(
            jax.ShapeDtypeStruct((rows, feats), jnp.float32),
            jax.ShapeDtypeStruct((rows,), jnp.float32),
        ),
        mesh=mesh,
        scratch_types=[
            pltpu.VMEM((2, rb, feats), jnp.float32),
            pltpu.VMEM((2, rb, feats), jnp.float32),
            pltpu.VMEM((rpw,), jnp.float32),
            pltpu.SemaphoreType.DMA,
            pltpu.SemaphoreType.DMA,
            pltpu.SemaphoreType.DMA,
            pltpu.SemaphoreType.DMA,
        ],
    )
    def rev_kernel(x_hbm, y_hbm, ld_hbm, in_v, out_v, zeros_v,
                   sin0, sin1, sout0, sout1):
        wid = lax.axis_index("s") * NC + lax.axis_index("c")
        base = wid * rpw
        sins = (sin0, sin1)
        souts = (sout0, sout1)

        zv = jnp.zeros((L,), jnp.float32)

        @plsc.parallel_loop(0, rpw // L)
        def _zfill(i):
            zeros_v[pl.ds(i * L, L)] = zv

        pltpu.sync_copy(zeros_v, ld_hbm.at[pl.ds(base, rpw)])

        def in_copy(g, b):
            return pltpu.async_copy(
                x_hbm.at[pl.ds(base + g * rb, rb)], in_v.at[b], sins[b])

        def out_copy(g, b):
            return pltpu.async_copy(
                out_v.at[b], y_hbm.at[pl.ds(base + g * rb, rb)], souts[b])

        in_copy(0, 0)

        @pl.loop(0, nb, step=2)
        def _blocks(g0):
            for b in range(2):
                g = g0 + b
                bn = (b + 1) % 2

                @pl.when(g + 1 < nb)
                def _prefetch():
                    in_copy(g + 1, bn)

                pltpu.make_async_copy(
                    x_hbm.at[pl.ds(base + g * rb, rb)],
                    in_v.at[b], sins[b]).wait()

                @pl.when(g >= 2)
                def _drain():
                    pltpu.make_async_copy(
                        out_v.at[b],
                        y_hbm.at[pl.ds(base + g * rb, rb)],
                        souts[b]).wait()

                for r in range(rb):
                    @plsc.parallel_loop(0, nch, unroll=8)
                    def _chunk(j):
                        v = in_v[b, r, pl.ds((nch - 1 - j) * L, L)]
                        out_v[b, r, pl.ds(j * L, L)] = lax.rev(v, (0,))

                out_copy(g, b)

        for b in range(2):
            pltpu.make_async_copy(
                out_v.at[b],
                y_hbm.at[pl.ds(base + (nb - 2 + b) * rb, rb)],
                souts[b]).wait()

    return rev_kernel


def kernel(x, perm):
    rows, feats = x.shape
    y, logdet = _build(rows, feats)(x)
    return (y, logdet)
